# baseline (speedup 1.0000x reference)
import dataclasses
import functools

import jax
import jax.numpy as jnp
from jax import lax
from jax.experimental import pallas as pl
from jax.experimental.pallas import tpu as pltpu
from jax.experimental.pallas import tpu_sc as plsc

ROWS = 128
VOCAB = 100000
LANES = 16
NWORK = 32
SC_COLT = 780
HALF_T = SC_COLT // 2
TAIL0 = SC_COLT * 128
TAIL = VOCAB - TAIL0
CHT = 30
NJOB = HALF_T // CHT
NBUF = 3
_INT_MAX = 2**31 - 1


def _merge(rm_a, ri_a, rm_b, ri_b):
    take_b = (rm_b > rm_a) | ((rm_b == rm_a) & (ri_b < ri_a))
    return jnp.where(take_b, rm_b, rm_a), jnp.where(take_b, ri_b, ri_a)


def _sc_scan(m_logits):
    mesh = plsc.VectorSubcoreMesh(core_axis_name="c", subcore_axis_name="s")

    scratch = [pltpu.VMEM((8, CHT * 128), jnp.float32) for _ in range(NBUF)]
    scratch.append(pltpu.VMEM((LANES,), jnp.float32))
    scratch.append(pltpu.VMEM((LANES,), jnp.int32))
    scratch.extend(pltpu.SemaphoreType.DMA for _ in range(NBUF))

    cp = pltpu.CompilerParams()
    cp = dataclasses.replace(cp, use_tc_tiling_on_sc=True)
    if "needs_layout_passes" in pltpu.CompilerParams.__dataclass_fields__:
        cp = dataclasses.replace(cp, needs_layout_passes=False)

    @functools.partial(
        pl.kernel,
        out_type=(
            jax.ShapeDtypeStruct((NWORK, LANES), jnp.float32),
            jax.ShapeDtypeStruct((NWORK, LANES), jnp.int32),
        ),
        mesh=mesh,
        scratch_types=scratch,
        compiler_params=cp,
    )
    def sc_argmax(x_hbm, val_hbm, idx_hbm, *rest):
        bufs = rest[:NBUF]
        outv_f = rest[NBUF]
        outv_i = rest[NBUF + 1]
        sems = rest[NBUF + 2:]

        w = lax.axis_index("c") * 16 + lax.axis_index("s")
        row0 = (w // 2) * 8
        colt0 = (w % 2) * HALF_T

        def issue(j):
            c0 = (colt0 + j * CHT) * 128
            return pltpu.async_copy(
                x_hbm.at[pl.ds(row0, 8), pl.ds(c0, CHT * 128)],
                bufs[j % NBUF],
                sems[j % NBUF],
            )

        copies = {}
        for j in range(min(NBUF - 1, NJOB)):
            copies[j] = issue(j)

        iota = lax.iota(jnp.int32, LANES)
        neg_inf = jnp.full((LANES,), -jnp.inf, dtype=jnp.float32)
        zero_i = jnp.zeros((LANES,), dtype=jnp.int32)

        rms = [neg_inf] * 8
        ris = [zero_i] * 8

        for j in range(NJOB):
            nxt = j + (NBUF - 1)
            if nxt < NJOB:
                copies[nxt] = issue(nxt)
            copies[j].wait()
            buf = bufs[j % NBUF]
            cbase = (colt0 + j * CHT) * 128

            def body(i, carry, buf=buf, cbase=cbase):
                c_rms, c_ris = carry
                c_rms, c_ris = list(c_rms), list(c_ris)
                iv = iota + (cbase + i * LANES)
                off = i * LANES
                for jr in range(8):
                    v = buf[jr, pl.ds(off, LANES)]
                    m = v > c_rms[jr]
                    c_rms[jr] = jnp.where(m, v, c_rms[jr])
                    c_ris[jr] = jnp.where(m, iv, c_ris[jr])
                return tuple(c_rms), tuple(c_ris)

            rms_t, ris_t = lax.fori_loop(
                0, CHT * 8, body, (tuple(rms), tuple(ris))
            )
            rms, ris = list(rms_t), list(ris_t)

        val_vec = jnp.zeros((LANES,), dtype=jnp.float32)
        idx_vec = jnp.zeros((LANES,), dtype=jnp.int32)
        for jr in range(8):
            row_max = jnp.max(rms[jr])
            cand = jnp.where(rms[jr] == row_max, ris[jr], jnp.int32(_INT_MAX))
            ans = jnp.min(cand)
            val_vec = jnp.where(iota == jr, row_max, val_vec)
            idx_vec = jnp.where(iota == jr, ans, idx_vec)

        outv_f[...] = val_vec
        outv_i[...] = idx_vec
        pltpu.sync_copy(outv_f, val_hbm.at[w])
        pltpu.sync_copy(outv_i, idx_hbm.at[w])

    return sc_argmax(m_logits)


def _tc_tail_merge(m_logits, sc_val, sc_idx):
    def body(x_ref, v_ref, i_ref, o_ref):
        x = x_ref[...]
        col = lax.broadcasted_iota(jnp.int32, x.shape, 1)
        valid = col < TAIL
        v = jnp.where(valid, x, -jnp.inf)
        m_t = jnp.max(v, axis=1, keepdims=True)
        cand = jnp.where(v == m_t, col + TAIL0, _INT_MAX)
        i_t = jnp.min(cand, axis=1, keepdims=True)

        rm, ri = v_ref[:, 0:1], i_ref[:, 0:1]
        rm, ri = _merge(rm, ri, v_ref[:, 1:2], i_ref[:, 1:2])
        _, ri = _merge(rm, ri, m_t, i_t)
        o_ref[...] = ri

    return pl.pallas_call(
        body,
        grid=(1,),
        out_shape=jax.ShapeDtypeStruct((ROWS, 1), jnp.int32),
        in_specs=[
            pl.BlockSpec((ROWS, 256), lambda i: (0, TAIL0 // 256)),
            pl.BlockSpec((ROWS, 2), lambda i: (0, 0)),
            pl.BlockSpec((ROWS, 2), lambda i: (0, 0)),
        ],
        out_specs=pl.BlockSpec((ROWS, 1), lambda i: (0, 0)),
    )(m_logits, sc_val, sc_idx)


def kernel(m_logits):
    val, idx = _sc_scan(m_logits)
    val2 = val.reshape(16, 2, LANES)[:, :, :8].transpose(0, 2, 1).reshape(ROWS, 2)
    idx2 = idx.reshape(16, 2, LANES)[:, :, :8].transpose(0, 2, 1).reshape(ROWS, 2)
    return _tc_tail_merge(m_logits, val2, idx2)


# speedup vs baseline: 2.1346x; 2.1346x over previous
import dataclasses
import functools

import jax
import jax.numpy as jnp
from jax import lax
from jax.experimental import pallas as pl
from jax.experimental.pallas import tpu as pltpu
from jax.experimental.pallas import tpu_sc as plsc

ROWS = 128
VOCAB = 100000
LANES = 16
NWORK = 32
CHC = 200
NCHUNK = VOCAB // CHC
NJOB = -(-NCHUNK // NWORK)
NBUF = 4
_INT_MAX = 2**31 - 1


def _sc_scan(xt):
    mesh = plsc.VectorSubcoreMesh(core_axis_name="c", subcore_axis_name="s")

    scratch = [pltpu.VMEM((CHC, ROWS), jnp.float32) for _ in range(NBUF)]
    scratch.append(pltpu.VMEM((ROWS,), jnp.float32))
    scratch.append(pltpu.VMEM((ROWS,), jnp.int32))
    scratch.extend(pltpu.SemaphoreType.DMA for _ in range(NBUF))
    n_full = NCHUNK - (NJOB - 1) * NWORK

    cp = pltpu.CompilerParams()
    if "needs_layout_passes" in pltpu.CompilerParams.__dataclass_fields__:
        cp = dataclasses.replace(cp, needs_layout_passes=False)

    @functools.partial(
        pl.kernel,
        out_type=(
            jax.ShapeDtypeStruct((NWORK, ROWS), jnp.float32),
            jax.ShapeDtypeStruct((NWORK, ROWS), jnp.int32),
        ),
        mesh=mesh,
        scratch_types=scratch,
        compiler_params=cp,
    )
    def sc_argmax(x_hbm, val_hbm, idx_hbm, *rest):
        bufs = rest[:NBUF]
        outv_f = rest[NBUF]
        outv_i = rest[NBUF + 1]
        sems = rest[NBUF + 2:]

        w = lax.axis_index("c") * 16 + lax.axis_index("s")
        has_last = w < n_full

        def _descr(j):
            off = pl.multiple_of((w + j * NWORK) * CHC, 8)
            return pltpu.make_async_copy(
                x_hbm.at[pl.ds(off, CHC), :],
                bufs[j % NBUF],
                sems[j % NBUF],
            )

        def issue(j):
            copy = _descr(j)
            copy.start()
            return copy

        def guarded_issue(j):
            if j < NJOB - 1:
                return issue(j)

            @pl.when(has_last)
            def _():
                issue(j)

        copies = {}
        for j in range(min(NBUF - 1, NJOB)):
            copies[j] = guarded_issue(j)

        neg_inf = jnp.full((LANES,), -jnp.inf, dtype=jnp.float32)
        zero_i = jnp.zeros((LANES,), dtype=jnp.int32)

        rms = [neg_inf] * 8
        ris = [zero_i] * 8

        for j in range(NJOB):
            nxt = j + (NBUF - 1)
            if nxt < NJOB:
                copies[nxt] = guarded_issue(nxt)
            last = j == NJOB - 1
            if not last:
                copies[j].wait()
            else:

                @pl.when(has_last)
                def _(j=j):
                    _descr(j).wait()

            buf = bufs[j % NBUF]
            cbase = (w + j * NWORK) * CHC
            madd = None
            if last:
                madd = jnp.where(
                    has_last, jnp.float32(0), jnp.float32(-jnp.inf)
                )

            def body(i, carry, buf=buf, cbase=cbase, madd=madd):
                c_rms, c_ris = carry
                c_rms, c_ris = list(c_rms), list(c_ris)
                col = jnp.broadcast_to(cbase + i, (LANES,)).astype(jnp.int32)
                for k in range(8):
                    v = buf[i, pl.ds(k * LANES, LANES)]
                    if madd is not None:
                        v = v + madd
                    m = v > c_rms[k]
                    c_rms[k] = jnp.where(m, v, c_rms[k])
                    c_ris[k] = jnp.where(m, col, c_ris[k])
                return tuple(c_rms), tuple(c_ris)

            rms_t, ris_t = lax.fori_loop(0, CHC, body, (tuple(rms), tuple(ris)))
            rms, ris = list(rms_t), list(ris_t)

        for k in range(8):
            outv_f[pl.ds(k * LANES, LANES)] = rms[k]
            outv_i[pl.ds(k * LANES, LANES)] = ris[k]
        pltpu.sync_copy(outv_f, val_hbm.at[w])
        pltpu.sync_copy(outv_i, idx_hbm.at[w])

    return sc_argmax(xt)


def _tc_merge(sc_val, sc_idx):
    def body(v_ref, i_ref, o_ref):
        v = v_ref[...]
        ix = i_ref[...]
        row_max = jnp.max(v, axis=0, keepdims=True)
        cand = jnp.where(v == row_max, ix, _INT_MAX)
        o_ref[...] = jnp.min(cand, axis=0, keepdims=True)

    return pl.pallas_call(
        body,
        out_shape=jax.ShapeDtypeStruct((1, ROWS), jnp.int32),
    )(sc_val, sc_idx)


def kernel(m_logits):
    val, idx = _sc_scan(m_logits.T)
    return _tc_merge(val, idx).reshape(ROWS, 1)


# speedup vs baseline: 2.1464x; 1.0055x over previous
import dataclasses
import functools

import jax
import jax.numpy as jnp
from jax import lax
from jax.experimental import pallas as pl
from jax.experimental.pallas import tpu as pltpu
from jax.experimental.pallas import tpu_sc as plsc

ROWS = 128
VOCAB = 100000
LANES = 16
NWORK = 32
CHC = 200
V_SC = 44000
NCHUNK = V_SC // CHC
NJOB = -(-NCHUNK // NWORK)
NBUF = 4
TCB = 2000
TC_BLOCKS = (VOCAB - V_SC) // TCB
_INT_MAX = 2**31 - 1


def _sc_scan(xt):
    mesh = plsc.VectorSubcoreMesh(core_axis_name="c", subcore_axis_name="s")

    scratch = [pltpu.VMEM((CHC, ROWS), jnp.float32) for _ in range(NBUF)]
    scratch.append(pltpu.VMEM((ROWS,), jnp.float32))
    scratch.append(pltpu.VMEM((ROWS,), jnp.int32))
    scratch.extend(pltpu.SemaphoreType.DMA for _ in range(NBUF))
    n_full = NCHUNK - (NJOB - 1) * NWORK

    cp = pltpu.CompilerParams()
    if "needs_layout_passes" in pltpu.CompilerParams.__dataclass_fields__:
        cp = dataclasses.replace(cp, needs_layout_passes=False)

    @functools.partial(
        pl.kernel,
        out_type=(
            jax.ShapeDtypeStruct((NWORK, ROWS), jnp.float32),
            jax.ShapeDtypeStruct((NWORK, ROWS), jnp.int32),
        ),
        mesh=mesh,
        scratch_types=scratch,
        compiler_params=cp,
    )
    def sc_argmax(x_hbm, val_hbm, idx_hbm, *rest):
        bufs = rest[:NBUF]
        outv_f = rest[NBUF]
        outv_i = rest[NBUF + 1]
        sems = rest[NBUF + 2:]

        w = lax.axis_index("c") * 16 + lax.axis_index("s")
        has_last = w < n_full

        def _descr(j):
            off = pl.multiple_of((w + j * NWORK) * CHC, 8)
            return pltpu.make_async_copy(
                x_hbm.at[pl.ds(off, CHC), :],
                bufs[j % NBUF],
                sems[j % NBUF],
            )

        def issue(j):
            copy = _descr(j)
            copy.start()
            return copy

        def guarded_issue(j):
            if j < NJOB - 1:
                return issue(j)

            @pl.when(has_last)
            def _():
                issue(j)

        copies = {}
        for j in range(min(NBUF - 1, NJOB)):
            copies[j] = guarded_issue(j)

        neg_inf = jnp.full((LANES,), -jnp.inf, dtype=jnp.float32)
        zero_i = jnp.zeros((LANES,), dtype=jnp.int32)

        rms = [neg_inf] * 8
        ris = [zero_i] * 8

        for j in range(NJOB):
            nxt = j + (NBUF - 1)
            if nxt < NJOB:
                copies[nxt] = guarded_issue(nxt)
            last = j == NJOB - 1
            if not last:
                copies[j].wait()
            else:

                @pl.when(has_last)
                def _(j=j):
                    _descr(j).wait()

            buf = bufs[j % NBUF]
            cbase = (w + j * NWORK) * CHC
            madd = None
            if last:
                madd = jnp.where(
                    has_last, jnp.float32(0), jnp.float32(-jnp.inf)
                )

            def body(i, carry, buf=buf, cbase=cbase, madd=madd):
                c_rms, c_ris = carry
                c_rms, c_ris = list(c_rms), list(c_ris)
                col = jnp.broadcast_to(cbase + i, (LANES,)).astype(jnp.int32)
                for k in range(8):
                    v = buf[i, pl.ds(k * LANES, LANES)]
                    if madd is not None:
                        v = v + madd
                    m = v > c_rms[k]
                    c_rms[k] = jnp.where(m, v, c_rms[k])
                    c_ris[k] = jnp.where(m, col, c_ris[k])
                return tuple(c_rms), tuple(c_ris)

            rms_t, ris_t = lax.fori_loop(0, CHC, body, (tuple(rms), tuple(ris)))
            rms, ris = list(rms_t), list(ris_t)

        for k in range(8):
            outv_f[pl.ds(k * LANES, LANES)] = rms[k]
            outv_i[pl.ds(k * LANES, LANES)] = ris[k]
        pltpu.sync_copy(outv_f, val_hbm.at[w])
        pltpu.sync_copy(outv_i, idx_hbm.at[w])

    return sc_argmax(xt)


def _tc_scan(xt):
    NACC = 4

    def body(x_ref, val_ref, idx_ref, *accs):
        pid = pl.program_id(0)
        av = accs[:NACC]
        ai = accs[NACC:]

        @pl.when(pid == 0)
        def _():
            for a in range(NACC):
                av[a][...] = jnp.full((8, ROWS), -jnp.inf, dtype=jnp.float32)
                ai[a][...] = jnp.zeros((8, ROWS), dtype=jnp.int32)

        x = x_ref[...]
        sub_iota = lax.broadcasted_iota(jnp.int32, (8, ROWS), 0)
        base = V_SC + pid * TCB
        cv = [av[a][...] for a in range(NACC)]
        ci = [ai[a][...] for a in range(NACC)]
        for j in range(TCB // 8):
            a = j % NACC
            xv = x[j * 8:(j + 1) * 8, :]
            iv = sub_iota + (base + j * 8)
            m = xv > cv[a]
            cv[a] = jnp.where(m, xv, cv[a])
            ci[a] = jnp.where(m, iv, ci[a])
        for a in range(NACC):
            av[a][...] = cv[a]
            ai[a][...] = ci[a]

        @pl.when(pid == TC_BLOCKS - 1)
        def _():
            rv, ri = cv[0], ci[0]
            for a in range(1, NACC):
                tb = (cv[a] > rv) | ((cv[a] == rv) & (ci[a] < ri))
                rv = jnp.where(tb, cv[a], rv)
                ri = jnp.where(tb, ci[a], ri)
            row_max = jnp.max(rv, axis=0, keepdims=True)
            cand = jnp.where(rv == row_max, ri, _INT_MAX)
            val_ref[...] = row_max
            idx_ref[...] = jnp.min(cand, axis=0, keepdims=True)

    return pl.pallas_call(
        body,
        grid=(TC_BLOCKS,),
        out_shape=(
            jax.ShapeDtypeStruct((1, ROWS), jnp.float32),
            jax.ShapeDtypeStruct((1, ROWS), jnp.int32),
        ),
        in_specs=[
            pl.BlockSpec((TCB, ROWS), lambda i: (V_SC // TCB + i, 0)),
        ],
        out_specs=(
            pl.BlockSpec((1, ROWS), lambda i: (0, 0)),
            pl.BlockSpec((1, ROWS), lambda i: (0, 0)),
        ),
        scratch_shapes=[pltpu.VMEM((8, ROWS), jnp.float32)] * NACC
        + [pltpu.VMEM((8, ROWS), jnp.int32)] * NACC,
    )(xt)


def _merge(sc_val, sc_idx, tc_val, tc_idx):
    def body(sv_ref, si_ref, tv_ref, ti_ref, o_ref):
        sv = sv_ref[...]
        si = si_ref[...]
        smax = jnp.max(sv, axis=0, keepdims=True)
        scand = jnp.min(
            jnp.where(sv == smax, si, _INT_MAX), axis=0, keepdims=True
        )
        tv = tv_ref[...]
        ti = ti_ref[...]
        tb = (tv > smax) | ((tv == smax) & (ti < scand))
        o_ref[...] = jnp.where(tb, ti, scand)

    return pl.pallas_call(
        body,
        out_shape=jax.ShapeDtypeStruct((1, ROWS), jnp.int32),
    )(sc_val, sc_idx, tc_val, tc_idx)


def kernel(m_logits):
    xt = m_logits.T
    sc_val, sc_idx = _sc_scan(xt)
    tc_val, tc_idx = _tc_scan(xt)
    return _merge(sc_val, sc_idx, tc_val, tc_idx).reshape(ROWS, 1)


# speedup vs baseline: 2.3646x; 1.1017x over previous
import dataclasses
import functools

import jax
import jax.numpy as jnp
from jax import lax
from jax.experimental import pallas as pl
from jax.experimental.pallas import tpu as pltpu
from jax.experimental.pallas import tpu_sc as plsc

ROWS = 128
VOCAB = 100000
LANES = 16
NWORK = 32
CHC = 200
V_SC = 56000
NCHUNK = V_SC // CHC
NJOB = -(-NCHUNK // NWORK)
NBUF = 4
TCB = 2000
TC_BLOCKS = (VOCAB - V_SC) // TCB
_INT_MAX = 2**31 - 1


def _sc_scan(xt):
    mesh = plsc.VectorSubcoreMesh(core_axis_name="c", subcore_axis_name="s")

    scratch = [pltpu.VMEM((CHC, ROWS), jnp.float32) for _ in range(NBUF)]
    scratch.append(pltpu.VMEM((ROWS,), jnp.float32))
    scratch.append(pltpu.VMEM((ROWS,), jnp.int32))
    scratch.extend(pltpu.SemaphoreType.DMA for _ in range(NBUF))
    n_full = NCHUNK - (NJOB - 1) * NWORK

    cp = pltpu.CompilerParams()
    if "needs_layout_passes" in pltpu.CompilerParams.__dataclass_fields__:
        cp = dataclasses.replace(cp, needs_layout_passes=False)

    @functools.partial(
        pl.kernel,
        out_type=(
            jax.ShapeDtypeStruct((NWORK, ROWS), jnp.float32),
            jax.ShapeDtypeStruct((NWORK, ROWS), jnp.int32),
        ),
        mesh=mesh,
        scratch_types=scratch,
        compiler_params=cp,
    )
    def sc_argmax(x_hbm, val_hbm, idx_hbm, *rest):
        bufs = rest[:NBUF]
        outv_f = rest[NBUF]
        outv_i = rest[NBUF + 1]
        sems = rest[NBUF + 2:]

        w = lax.axis_index("c") * 16 + lax.axis_index("s")
        has_last = w < n_full

        def _descr(j):
            off = pl.multiple_of((w + j * NWORK) * CHC, 8)
            return pltpu.make_async_copy(
                x_hbm.at[pl.ds(off, CHC), :],
                bufs[j % NBUF],
                sems[j % NBUF],
            )

        def issue(j):
            copy = _descr(j)
            copy.start()
            return copy

        def guarded_issue(j):
            if j < NJOB - 1:
                return issue(j)

            @pl.when(has_last)
            def _():
                issue(j)

        copies = {}
        for j in range(min(NBUF - 1, NJOB)):
            copies[j] = guarded_issue(j)

        neg_inf = jnp.full((LANES,), -jnp.inf, dtype=jnp.float32)
        zero_i = jnp.zeros((LANES,), dtype=jnp.int32)

        rms = [neg_inf] * 8
        ris = [zero_i] * 8

        for j in range(NJOB):
            nxt = j + (NBUF - 1)
            if nxt < NJOB:
                copies[nxt] = guarded_issue(nxt)
            last = j == NJOB - 1
            if not last:
                copies[j].wait()
            else:

                @pl.when(has_last)
                def _(j=j):
                    _descr(j).wait()

            buf = bufs[j % NBUF]
            cbase = (w + j * NWORK) * CHC
            madd = None
            if last:
                madd = jnp.where(
                    has_last, jnp.float32(0), jnp.float32(-jnp.inf)
                )

            def body(i, carry, buf=buf, cbase=cbase, madd=madd):
                c_rms, c_ris = carry
                c_rms, c_ris = list(c_rms), list(c_ris)
                col = jnp.broadcast_to(cbase + i, (LANES,)).astype(jnp.int32)
                for k in range(8):
                    v = buf[i, pl.ds(k * LANES, LANES)]
                    if madd is not None:
                        v = v + madd
                    m = v > c_rms[k]
                    c_rms[k] = jnp.where(m, v, c_rms[k])
                    c_ris[k] = jnp.where(m, col, c_ris[k])
                return tuple(c_rms), tuple(c_ris)

            rms_t, ris_t = lax.fori_loop(0, CHC, body, (tuple(rms), tuple(ris)))
            rms, ris = list(rms_t), list(ris_t)

        for k in range(8):
            outv_f[pl.ds(k * LANES, LANES)] = rms[k]
            outv_i[pl.ds(k * LANES, LANES)] = ris[k]
        pltpu.sync_copy(outv_f, val_hbm.at[w])
        pltpu.sync_copy(outv_i, idx_hbm.at[w])

    return sc_argmax(xt)


def _tc_scan(xt):
    NACC = 4

    def body(x_ref, val_ref, idx_ref, *accs):
        pid = pl.program_id(0)
        av = accs[:NACC]
        ai = accs[NACC:]

        @pl.when(pid == 0)
        def _():
            for a in range(NACC):
                av[a][...] = jnp.full((8, ROWS), -jnp.inf, dtype=jnp.float32)
                ai[a][...] = jnp.zeros((8, ROWS), dtype=jnp.int32)

        x = x_ref[...]
        sub_iota = lax.broadcasted_iota(jnp.int32, (8, ROWS), 0)
        base = V_SC + pid * TCB
        cv = [av[a][...] for a in range(NACC)]
        ci = [ai[a][...] for a in range(NACC)]
        for j in range(TCB // 8):
            a = j % NACC
            xv = x[j * 8:(j + 1) * 8, :]
            iv = sub_iota + (base + j * 8)
            m = xv > cv[a]
            cv[a] = jnp.where(m, xv, cv[a])
            ci[a] = jnp.where(m, iv, ci[a])
        for a in range(NACC):
            av[a][...] = cv[a]
            ai[a][...] = ci[a]

        @pl.when(pid == TC_BLOCKS - 1)
        def _():
            rv, ri = cv[0], ci[0]
            for a in range(1, NACC):
                tb = (cv[a] > rv) | ((cv[a] == rv) & (ci[a] < ri))
                rv = jnp.where(tb, cv[a], rv)
                ri = jnp.where(tb, ci[a], ri)
            row_max = jnp.max(rv, axis=0, keepdims=True)
            cand = jnp.where(rv == row_max, ri, _INT_MAX)
            val_ref[...] = row_max
            idx_ref[...] = jnp.min(cand, axis=0, keepdims=True)

    return pl.pallas_call(
        body,
        grid=(TC_BLOCKS,),
        out_shape=(
            jax.ShapeDtypeStruct((1, ROWS), jnp.float32),
            jax.ShapeDtypeStruct((1, ROWS), jnp.int32),
        ),
        in_specs=[
            pl.BlockSpec((TCB, ROWS), lambda i: (V_SC // TCB + i, 0)),
        ],
        out_specs=(
            pl.BlockSpec((1, ROWS), lambda i: (0, 0)),
            pl.BlockSpec((1, ROWS), lambda i: (0, 0)),
        ),
        scratch_shapes=[pltpu.VMEM((8, ROWS), jnp.float32)] * NACC
        + [pltpu.VMEM((8, ROWS), jnp.int32)] * NACC,
    )(xt)


def _merge(sc_val, sc_idx, tc_val, tc_idx):
    def body(sv_ref, si_ref, tv_ref, ti_ref, o_ref):
        sv = sv_ref[...]
        si = si_ref[...]
        smax = jnp.max(sv, axis=0, keepdims=True)
        scand = jnp.min(
            jnp.where(sv == smax, si, _INT_MAX), axis=0, keepdims=True
        )
        tv = tv_ref[...]
        ti = ti_ref[...]
        tb = (tv > smax) | ((tv == smax) & (ti < scand))
        o_ref[...] = jnp.where(tb, ti, scand)

    return pl.pallas_call(
        body,
        out_shape=jax.ShapeDtypeStruct((1, ROWS), jnp.int32),
    )(sc_val, sc_idx, tc_val, tc_idx)


def kernel(m_logits):
    xt = m_logits.T
    sc_val, sc_idx = _sc_scan(xt)
    tc_val, tc_idx = _tc_scan(xt)
    return _merge(sc_val, sc_idx, tc_val, tc_idx).reshape(ROWS, 1)
